# baseline (speedup 1.0000x reference)
import functools

import jax
import jax.numpy as jnp
from jax import lax
from jax.experimental import pallas as pl
from jax.experimental.pallas import tpu as pltpu
from jax.experimental.pallas import tpu_sc as plsc

EMBED = 1024
CHUNK = 80
PCHUNK = 80


def _dense_body(x0_ref, word_ref, wih_ref, bih_ref, bhh_ref, len_ref,
                obj_ref, linw_ref, linb_ref, rel_ref,
                cap_ref, proj_ref, pair_ref, xe_s, csem):
    b = cap_ref.shape[0]

    def cp(i, carry):
        pltpu.make_async_copy(word_ref.at[pl.ds(x0_ref[i], 1), :],
                              xe_s.at[pl.ds(i, 1), :], csem).start()
        return carry

    lax.fori_loop(0, b, cp, 0)

    def wt(i, carry):
        pltpu.make_async_copy(word_ref.at[pl.ds(x0_ref[i], 1), :],
                              xe_s.at[pl.ds(i, 1), :], csem).wait()
        return carry

    lax.fori_loop(0, b, wt, 0)
    gi = lax.dot_general(xe_s[...], wih_ref[...], (((1,), (1,)), ((), ())),
                         preferred_element_type=jnp.float32) + bih_ref[...]
    bhh = bhh_ref[...]
    i_r = gi[:, :EMBED]
    i_z = gi[:, EMBED:2 * EMBED]
    i_n = gi[:, 2 * EMBED:]
    h_r = bhh[:, :EMBED]
    h_z = bhh[:, EMBED:2 * EMBED]
    h_n = bhh[:, 2 * EMBED:]
    r = jax.nn.sigmoid(i_r + h_r)
    z = jax.nn.sigmoid(i_z + h_z)
    n = jnp.tanh(i_n + r * h_n)
    h_new = (1.0 - z) * n
    mask = 0 < len_ref[...]
    out = jnp.where(mask, h_new, 0.0)
    norm = jnp.sqrt(jnp.sum(out * out, axis=1, keepdims=True)) + 1e-8
    cap_ref[...] = out / norm
    proj_ref[...] = lax.dot_general(obj_ref[...], linw_ref[...],
                                    (((1,), (1,)), ((), ())),
                                    preferred_element_type=jnp.float32) \
        + linb_ref[...]
    rel = rel_ref[...]
    nv, dp = rel.shape
    pair_ref[...] = jnp.concatenate(
        [jnp.broadcast_to(rel[:, None, :], (nv, nv, dp)),
         jnp.broadcast_to(rel[None, :, :], (nv, nv, dp)),
         jnp.zeros((nv, nv, 8), jnp.float32)], axis=2)


def _dense_tc(x0, word_embed, W_ih, b_ih, b_hh, lengths, obj_embed,
              lin_W, lin_b, rel_embed):
    b = x0.shape[0]
    d = word_embed.shape[1]
    nobj = obj_embed.shape[0]
    gconv = lin_W.shape[0]
    nv, dp = rel_embed.shape
    return pl.pallas_call(
        _dense_body,
        in_specs=[pl.BlockSpec(memory_space=pltpu.SMEM),
                  pl.BlockSpec(memory_space=pl.ANY)]
        + [pl.BlockSpec(memory_space=pltpu.VMEM)] * 8,
        out_shape=(jax.ShapeDtypeStruct((b, EMBED), jnp.float32),
                   jax.ShapeDtypeStruct((nobj, gconv), jnp.float32),
                   jax.ShapeDtypeStruct((nv, nv, 2 * dp + 8), jnp.float32)),
        scratch_shapes=[pltpu.VMEM((b, d), jnp.float32),
                        pltpu.SemaphoreType.DMA],
    )(x0, word_embed, W_ih, b_ih.reshape(1, -1), b_hh.reshape(1, -1),
      lengths.reshape(b, 1), obj_embed, lin_W, lin_b.reshape(1, -1), rel_embed)


def _sc_pred(p_idx, pair_table, nv, dp):
    e2 = p_idx.shape[0] // 2
    dpair = 2 * dp
    dpad = pair_table.shape[1]
    info = plsc.get_sparse_core_info()
    nw = info.num_cores * info.num_subcores
    n_pred_chunks = e2 // PCHUNK
    pred_iters = -(-n_pred_chunks // nw)
    mesh = plsc.VectorSubcoreMesh(core_axis_name="c", subcore_axis_name="s")
    pred_pair_iters = -(-pred_iters // 2)

    @functools.partial(
        pl.kernel, mesh=mesh,
        compiler_params=pltpu.CompilerParams(use_tc_tiling_on_sc=False,
                                             needs_layout_passes=False),
        out_type=jax.ShapeDtypeStruct((e2, dpair), jnp.float32),
        scratch_types=[
            pltpu.VMEM((2 * PCHUNK,), jnp.int32),
            pltpu.VMEM((PCHUNK,), jnp.int32),
            pltpu.VMEM((PCHUNK, dpad), jnp.float32),
            pltpu.VMEM((PCHUNK, dpad), jnp.float32),
            pltpu.SemaphoreType.DMA,
            pltpu.SemaphoreType.DMA,
            pltpu.SemaphoreType.DMA,
        ],
    )
    def k(pidx_hbm, pair_hbm, pred_out,
          p1d_v, pidx_v, prow0, prow1, gsem, wsem0, wsem1):
        wid = lax.axis_index("s") * info.num_cores + lax.axis_index("c")
        iot = lax.iota(jnp.int32, 16)

        def pred_sub(it2, bsel, rbuf, wsem):
            chunk = (it2 * 2 + bsel) * nw + wid

            @pl.when(chunk < n_pred_chunks)
            def _():
                base = chunk * PCHUNK
                pltpu.sync_copy(pidx_hbm.at[pl.ds(2 * base, 2 * PCHUNK)],
                                p1d_v)
                for g in range(PCHUNK // 16):
                    rows = (g * 16 + iot) * 2
                    ev = plsc.load_gather(p1d_v, [rows])
                    od = plsc.load_gather(p1d_v, [rows + 1])
                    pidx_v[pl.ds(g * 16, 16)] = ev * nv + od

                @pl.when(it2 > 0)
                def _drain():
                    pltpu.make_async_copy(rbuf.at[:, pl.ds(0, dpair)],
                                          pred_out.at[pl.ds(base, PCHUNK)],
                                          wsem).wait()
                pltpu.async_copy(pair_hbm.at[pidx_v], rbuf, gsem).wait()
                pltpu.async_copy(rbuf.at[:, pl.ds(0, dpair)],
                                 pred_out.at[pl.ds(base, PCHUNK)], wsem)

        def pred_body(it2, carry):
            pred_sub(it2, 0, prow0, wsem0)
            pred_sub(it2, 1, prow1, wsem1)
            return carry

        lax.fori_loop(0, pred_pair_iters, pred_body, 0)
        pltpu.make_async_copy(prow0.at[:, pl.ds(0, dpair)],
                              pred_out.at[pl.ds(0, PCHUNK)], wsem0).wait()
        pltpu.make_async_copy(prow1.at[:, pl.ds(0, dpair)],
                              pred_out.at[pl.ds(0, PCHUNK)], wsem1).wait()

    return k(p_idx, pair_table)


def _sc_obj(obj_idx, proj):
    o = obj_idx.shape[0]
    do = proj.shape[1]
    info = plsc.get_sparse_core_info()
    nw = info.num_cores * info.num_subcores
    n_obj_chunks = o // CHUNK
    obj_iters = -(-n_obj_chunks // nw)
    mesh = plsc.VectorSubcoreMesh(core_axis_name="c", subcore_axis_name="s")
    obj_pair_iters = -(-obj_iters // 2)

    @functools.partial(
        pl.kernel, mesh=mesh,
        compiler_params=pltpu.CompilerParams(use_tc_tiling_on_sc=False,
                                             needs_layout_passes=False),
        out_type=jax.ShapeDtypeStruct((o, do), jnp.float32),
        scratch_types=[
            pltpu.VMEM((CHUNK,), jnp.int32),
            pltpu.VMEM((CHUNK, do), jnp.float32),
            pltpu.VMEM((CHUNK, do), jnp.float32),
            pltpu.SemaphoreType.DMA,
            pltpu.SemaphoreType.DMA,
            pltpu.SemaphoreType.DMA,
        ],
    )
    def kobj(oidx_hbm, proj_hbm, obj_out,
             oidx_v, orow0, orow1, gsem, wsem0, wsem1):
        wid = lax.axis_index("s") * info.num_cores + lax.axis_index("c")

        def obj_sub(it2, bsel, rbuf, wsem):
            chunk = (it2 * 2 + bsel) * nw + wid

            @pl.when(chunk < n_obj_chunks)
            def _():
                base = chunk * CHUNK
                pltpu.sync_copy(oidx_hbm.at[pl.ds(base, CHUNK)], oidx_v)

                @pl.when(it2 > 0)
                def _drain():
                    pltpu.make_async_copy(rbuf,
                                          obj_out.at[pl.ds(base, CHUNK)],
                                          wsem).wait()
                pltpu.async_copy(proj_hbm.at[oidx_v], rbuf, gsem).wait()
                pltpu.async_copy(rbuf, obj_out.at[pl.ds(base, CHUNK)], wsem)

        def obj_body(it2, carry):
            obj_sub(it2, 0, orow0, wsem0)
            obj_sub(it2, 1, orow1, wsem1)
            return carry

        lax.fori_loop(0, obj_pair_iters, obj_body, 0)
        pltpu.make_async_copy(orow0, obj_out.at[pl.ds(0, CHUNK)], wsem0).wait()
        pltpu.make_async_copy(orow1, obj_out.at[pl.ds(0, CHUNK)], wsem1).wait()

    return kobj(obj_idx, proj)


def _untile_body(in_ref, out_ref):
    out_ref[...] = in_ref[...].reshape(out_ref.shape)


def _untile_tc(flat, e, dp, rows_per_block):
    nblk = e // rows_per_block
    in_rows = rows_per_block * dp // 128
    in3 = flat.reshape(nblk, in_rows, 128)
    return pl.pallas_call(
        _untile_body,
        grid=(nblk,),
        in_specs=[pl.BlockSpec((1, in_rows, 128), lambda i: (i, 0, 0))],
        out_specs=pl.BlockSpec((rows_per_block, dp), lambda i: (i, 0)),
        out_shape=jax.ShapeDtypeStruct((e, dp), jnp.float32),
    )(in3)


def kernel(x, lengths, cap_obj_nums, cap_pred_nums, cap_obj_list, cap_rel_list,
           word_embed, W_ih, W_hh, b_ih, b_hh, obj_embed, rel_embed,
           lin_W, lin_b):
    del cap_obj_nums, cap_pred_nums, W_hh
    b = x.shape[0]
    e = cap_rel_list.shape[0]
    nv, dp = rel_embed.shape
    x0 = x[:, 0]
    p_idx = cap_rel_list[:, 1]
    cap, proj, pair3 = _dense_tc(x0, word_embed, W_ih, b_ih, b_hh, lengths,
                                 obj_embed, lin_W, lin_b, rel_embed)
    pair_table = pair3.reshape(nv * nv, 2 * dp + 8)
    eh = e // 2
    pred2a = _sc_pred(p_idx[:eh], pair_table, nv, dp)
    pred2b = _sc_pred(p_idx[eh:], pair_table, nv, dp)
    obj_vecs = _sc_obj(cap_obj_list, proj)
    pred_vecs = jnp.concatenate(
        [pred2a.reshape(eh, dp), pred2b.reshape(eh, dp)], axis=0)
    cap_emb = cap.reshape(b, 1, EMBED)
    return (cap_emb, lengths, obj_vecs, pred_vecs)


# speedup vs baseline: 1.0713x; 1.0713x over previous
import functools

import jax
import jax.numpy as jnp
from jax import lax
from jax.experimental import pallas as pl
from jax.experimental.pallas import tpu as pltpu
from jax.experimental.pallas import tpu_sc as plsc

EMBED = 1024
CHUNK = 80
PCHUNK = 80


def _dense_body(x0_ref, word_ref, wih_ref, bih_ref, bhh_ref, len_ref,
                obj_ref, linw_ref, linb_ref, rel_ref,
                cap_ref, proj_ref, pair_ref, xe_s, csem):
    b = cap_ref.shape[0]

    def cp(i, carry):
        pltpu.make_async_copy(word_ref.at[pl.ds(x0_ref[i], 1), :],
                              xe_s.at[pl.ds(i, 1), :], csem).start()
        return carry

    lax.fori_loop(0, b, cp, 0)

    def wt(i, carry):
        pltpu.make_async_copy(word_ref.at[pl.ds(x0_ref[i], 1), :],
                              xe_s.at[pl.ds(i, 1), :], csem).wait()
        return carry

    lax.fori_loop(0, b, wt, 0)
    gi = lax.dot_general(xe_s[...], wih_ref[...], (((1,), (1,)), ((), ())),
                         preferred_element_type=jnp.float32) + bih_ref[...]
    bhh = bhh_ref[...]
    i_r = gi[:, :EMBED]
    i_z = gi[:, EMBED:2 * EMBED]
    i_n = gi[:, 2 * EMBED:]
    h_r = bhh[:, :EMBED]
    h_z = bhh[:, EMBED:2 * EMBED]
    h_n = bhh[:, 2 * EMBED:]
    r = jax.nn.sigmoid(i_r + h_r)
    z = jax.nn.sigmoid(i_z + h_z)
    n = jnp.tanh(i_n + r * h_n)
    h_new = (1.0 - z) * n
    mask = 0 < len_ref[...]
    out = jnp.where(mask, h_new, 0.0)
    norm = jnp.sqrt(jnp.sum(out * out, axis=1, keepdims=True)) + 1e-8
    cap_ref[...] = out / norm
    proj_ref[...] = lax.dot_general(obj_ref[...], linw_ref[...],
                                    (((1,), (1,)), ((), ())),
                                    preferred_element_type=jnp.float32) \
        + linb_ref[...]
    rel = rel_ref[...]
    nv, dp = rel.shape
    pair_ref[...] = jnp.concatenate(
        [jnp.broadcast_to(rel[:, None, :], (nv, nv, dp)),
         jnp.broadcast_to(rel[None, :, :], (nv, nv, dp)),
         jnp.zeros((nv, nv, 8), jnp.float32)], axis=2)


def _dense_tc(x0, word_embed, W_ih, b_ih, b_hh, lengths, obj_embed,
              lin_W, lin_b, rel_embed):
    b = x0.shape[0]
    d = word_embed.shape[1]
    nobj = obj_embed.shape[0]
    gconv = lin_W.shape[0]
    nv, dp = rel_embed.shape
    return pl.pallas_call(
        _dense_body,
        in_specs=[pl.BlockSpec(memory_space=pltpu.SMEM),
                  pl.BlockSpec(memory_space=pl.ANY)]
        + [pl.BlockSpec(memory_space=pltpu.VMEM)] * 8,
        out_shape=(jax.ShapeDtypeStruct((b, EMBED), jnp.float32),
                   jax.ShapeDtypeStruct((nobj, gconv), jnp.float32),
                   jax.ShapeDtypeStruct((nv, nv, 2 * dp + 8), jnp.float32)),
        scratch_shapes=[pltpu.VMEM((b, d), jnp.float32),
                        pltpu.SemaphoreType.DMA],
    )(x0, word_embed, W_ih, b_ih.reshape(1, -1), b_hh.reshape(1, -1),
      lengths.reshape(b, 1), obj_embed, lin_W, lin_b.reshape(1, -1), rel_embed)


def _sc_pred(p_idx, pair_table, nv, dp):
    e2 = p_idx.shape[0] // 2
    dpair = 2 * dp
    dpad = pair_table.shape[1]
    info = plsc.get_sparse_core_info()
    nw = info.num_cores * info.num_subcores
    n_pred_chunks = e2 // PCHUNK
    pred_iters = -(-n_pred_chunks // nw)
    mesh = plsc.VectorSubcoreMesh(core_axis_name="c", subcore_axis_name="s")
    pred_pair_iters = -(-pred_iters // 2)

    @functools.partial(
        pl.kernel, mesh=mesh,
        compiler_params=pltpu.CompilerParams(use_tc_tiling_on_sc=False,
                                             needs_layout_passes=False),
        out_type=jax.ShapeDtypeStruct((e2, dpair), jnp.float32),
        scratch_types=[
            pltpu.VMEM((2 * PCHUNK,), jnp.int32),
            pltpu.VMEM((PCHUNK,), jnp.int32),
            pltpu.VMEM((PCHUNK, dpad), jnp.float32),
            pltpu.VMEM((PCHUNK, dpad), jnp.float32),
            pltpu.SemaphoreType.DMA,
            pltpu.SemaphoreType.DMA,
            pltpu.SemaphoreType.DMA,
        ],
    )
    def k(pidx_hbm, pair_hbm, pred_out,
          p1d_v, pidx_v, prow0, prow1, gsem, wsem0, wsem1):
        wid = lax.axis_index("s") * info.num_cores + lax.axis_index("c")
        iot = lax.iota(jnp.int32, 16)

        def pred_sub(it2, bsel, rbuf, wsem):
            chunk = (it2 * 2 + bsel) * nw + wid

            @pl.when(chunk < n_pred_chunks)
            def _():
                base = chunk * PCHUNK
                pltpu.sync_copy(pidx_hbm.at[pl.ds(2 * base, 2 * PCHUNK)],
                                p1d_v)
                for g in range(PCHUNK // 16):
                    rows = (g * 16 + iot) * 2
                    ev = plsc.load_gather(p1d_v, [rows])
                    od = plsc.load_gather(p1d_v, [rows + 1])
                    pidx_v[pl.ds(g * 16, 16)] = ev * nv + od

                @pl.when(it2 > 0)
                def _drain():
                    pltpu.make_async_copy(rbuf.at[:, pl.ds(0, dpair)],
                                          pred_out.at[pl.ds(base, PCHUNK)],
                                          wsem).wait()
                pltpu.async_copy(pair_hbm.at[pidx_v], rbuf, gsem).wait()
                pltpu.async_copy(rbuf.at[:, pl.ds(0, dpair)],
                                 pred_out.at[pl.ds(base, PCHUNK)], wsem)

        def pred_body(it2, carry):
            pred_sub(it2, 0, prow0, wsem0)
            pred_sub(it2, 1, prow1, wsem1)
            return carry

        lax.fori_loop(0, pred_pair_iters, pred_body, 0)
        pltpu.make_async_copy(prow0.at[:, pl.ds(0, dpair)],
                              pred_out.at[pl.ds(0, PCHUNK)], wsem0).wait()
        pltpu.make_async_copy(prow1.at[:, pl.ds(0, dpair)],
                              pred_out.at[pl.ds(0, PCHUNK)], wsem1).wait()

    return k(p_idx, pair_table)


def _sc_obj(obj_idx, proj):
    o = obj_idx.shape[0]
    do = proj.shape[1]
    info = plsc.get_sparse_core_info()
    nw = info.num_cores * info.num_subcores
    n_obj_chunks = o // CHUNK
    obj_iters = -(-n_obj_chunks // nw)
    mesh = plsc.VectorSubcoreMesh(core_axis_name="c", subcore_axis_name="s")
    obj_pair_iters = -(-obj_iters // 2)

    @functools.partial(
        pl.kernel, mesh=mesh,
        compiler_params=pltpu.CompilerParams(use_tc_tiling_on_sc=False,
                                             needs_layout_passes=False),
        out_type=jax.ShapeDtypeStruct((o, do), jnp.float32),
        scratch_types=[
            pltpu.VMEM((CHUNK,), jnp.int32),
            pltpu.VMEM((CHUNK, do), jnp.float32),
            pltpu.VMEM((CHUNK, do), jnp.float32),
            pltpu.SemaphoreType.DMA,
            pltpu.SemaphoreType.DMA,
            pltpu.SemaphoreType.DMA,
        ],
    )
    def kobj(oidx_hbm, proj_hbm, obj_out,
             oidx_v, orow0, orow1, gsem, wsem0, wsem1):
        wid = lax.axis_index("s") * info.num_cores + lax.axis_index("c")

        def obj_sub(it2, bsel, rbuf, wsem):
            chunk = (it2 * 2 + bsel) * nw + wid

            @pl.when(chunk < n_obj_chunks)
            def _():
                base = chunk * CHUNK
                pltpu.sync_copy(oidx_hbm.at[pl.ds(base, CHUNK)], oidx_v)

                @pl.when(it2 > 0)
                def _drain():
                    pltpu.make_async_copy(rbuf,
                                          obj_out.at[pl.ds(base, CHUNK)],
                                          wsem).wait()
                pltpu.async_copy(proj_hbm.at[oidx_v], rbuf, gsem).wait()
                pltpu.async_copy(rbuf, obj_out.at[pl.ds(base, CHUNK)], wsem)

        def obj_body(it2, carry):
            obj_sub(it2, 0, orow0, wsem0)
            obj_sub(it2, 1, orow1, wsem1)
            return carry

        lax.fori_loop(0, obj_pair_iters, obj_body, 0)
        pltpu.make_async_copy(orow0, obj_out.at[pl.ds(0, CHUNK)], wsem0).wait()
        pltpu.make_async_copy(orow1, obj_out.at[pl.ds(0, CHUNK)], wsem1).wait()

    return kobj(obj_idx, proj)


def _untile_body(in_ref, out_ref):
    out_ref[...] = in_ref[...].reshape(out_ref.shape)


def _untile_tc(flat, e, dp, rows_per_block):
    nblk = e // rows_per_block
    in_rows = rows_per_block * dp // 128
    in3 = flat.reshape(nblk, in_rows, 128)
    return pl.pallas_call(
        _untile_body,
        grid=(nblk,),
        in_specs=[pl.BlockSpec((1, in_rows, 128), lambda i: (i, 0, 0))],
        out_specs=pl.BlockSpec((rows_per_block, dp), lambda i: (i, 0)),
        out_shape=jax.ShapeDtypeStruct((e, dp), jnp.float32),
    )(in3)


def kernel(x, lengths, cap_obj_nums, cap_pred_nums, cap_obj_list, cap_rel_list,
           word_embed, W_ih, W_hh, b_ih, b_hh, obj_embed, rel_embed,
           lin_W, lin_b):
    del cap_obj_nums, cap_pred_nums, W_hh
    b = x.shape[0]
    e = cap_rel_list.shape[0]
    nv, dp = rel_embed.shape
    x0 = x[:, 0]
    p_idx = cap_rel_list[:, 1]
    cap, proj, pair3 = _dense_tc(x0, word_embed, W_ih, b_ih, b_hh, lengths,
                                 obj_embed, lin_W, lin_b, rel_embed)
    pair_table = pair3.reshape(nv * nv, 2 * dp + 8)
    pred2 = _sc_pred(p_idx, pair_table, nv, dp)
    obj_vecs = _sc_obj(cap_obj_list, proj)
    pred_vecs = pred2.reshape(e, dp)
    cap_emb = cap.reshape(b, 1, EMBED)
    return (cap_emb, lengths, obj_vecs, pred_vecs)
